# Initial kernel scaffold; baseline (speedup 1.0000x reference)
import jax
import jax.numpy as jnp
from jax.experimental import pallas as pl


def kernel(voc_idxs, pai_concept, pai_character):
    raise NotImplementedError("write your pallas kernel here")


# speedup vs baseline: 1.7760x; 1.7760x over previous
import functools

import jax
import jax.numpy as jnp
from jax import lax
from jax.experimental import pallas as pl
from jax.experimental.pallas import tpu as pltpu
from jax.experimental.pallas import tpu_sc as plsc

VOC = 100000
D = 16
K = 32
B = 4096
ROW = D * K

NUM_WORKERS = 32
B_PER_W = B // NUM_WORKERS


def _argmax_pass(rows_v, acc_v, add: bool):
    lanes = lax.iota(jnp.int32, 16)
    col0 = lanes * K

    def body(b, _):
        row_i = jnp.zeros((16,), jnp.int32) + b
        m = plsc.load_gather(rows_v, [row_i, col0])
        am = jnp.zeros((16,), jnp.int32)
        for k in range(1, K):
            v = plsc.load_gather(rows_v, [row_i, col0 + k])
            gt = v > m
            am = jnp.where(gt, k, am)
            m = jnp.where(gt, v, m)
        off = pl.multiple_of(b * D, D)
        if add:
            acc_v[pl.ds(off, D)] = acc_v[pl.ds(off, D)] + am
        else:
            acc_v[pl.ds(off, D)] = am
        return ()

    lax.fori_loop(0, B_PER_W, body, ())


@functools.partial(
    pl.kernel,
    out_type=jax.ShapeDtypeStruct((B * D,), jnp.int32),
    mesh=plsc.VectorSubcoreMesh(core_axis_name="c", subcore_axis_name="s"),
    compiler_params=pltpu.CompilerParams(
        use_tc_tiling_on_sc=False, needs_layout_passes=False
    ),
    scratch_types=[
        pltpu.VMEM((B_PER_W,), jnp.int32),
        pltpu.VMEM((B_PER_W, ROW), jnp.float32),
        pltpu.VMEM((B_PER_W * D,), jnp.int32),
        pltpu.SemaphoreType.DMA,
    ],
)
def _kd_encode(idx_hbm, concept_hbm, char_hbm, out_hbm, idx_v, rows_v, acc_v, sem):
    c = lax.axis_index("c")
    s = lax.axis_index("s")
    wid = s * 2 + c
    base = pl.multiple_of(wid * B_PER_W, B_PER_W)

    pltpu.sync_copy(idx_hbm.at[pl.ds(base, B_PER_W)], idx_v)

    pltpu.async_copy(concept_hbm.at[idx_v], rows_v, sem).wait()
    _argmax_pass(rows_v, acc_v, add=False)

    pltpu.async_copy(char_hbm.at[idx_v], rows_v, sem).wait()
    _argmax_pass(rows_v, acc_v, add=True)

    pltpu.sync_copy(acc_v, out_hbm.at[pl.ds(base * D, B_PER_W * D)])


def kernel(voc_idxs, pai_concept, pai_character):
    idx = voc_idxs.astype(jnp.int32)
    concept = pai_concept.reshape(VOC, ROW)
    character = pai_character.reshape(VOC, ROW)
    out = _kd_encode(idx, concept, character)
    return out.reshape(B, D)


# speedup vs baseline: 5.5992x; 3.1527x over previous
import functools

import jax
import jax.numpy as jnp
from jax import lax
from jax.experimental import pallas as pl
from jax.experimental.pallas import tpu as pltpu
from jax.experimental.pallas import tpu_sc as plsc

VOC = 100000
D = 16
K = 32
B = 4096


VB = 2048
GRID = (VOC + VB - 1) // VB


def _tc_body(cref, chref, oref):
    def table_argmax(ref):
        m = ref[:, 0, :]
        am = jnp.zeros(m.shape, jnp.int32)
        for k in range(1, K):
            v = ref[:, k, :]
            gt = v > m
            am = jnp.where(gt, k, am)
            m = jnp.where(gt, v, m)
        return am

    oref[...] = table_argmax(cref) + table_argmax(chref)


def _tc_scan(ct, cht):
    return pl.pallas_call(
        _tc_body,
        grid=(GRID,),
        in_specs=[
            pl.BlockSpec((D, K, VB), lambda i: (0, 0, i)),
            pl.BlockSpec((D, K, VB), lambda i: (0, 0, i)),
        ],
        out_specs=pl.BlockSpec((D, VB), lambda i: (0, i)),
        out_shape=jax.ShapeDtypeStruct((D, VOC), jnp.int32),
    )(ct, cht)


NUM_WORKERS = 32
B_PER_W = B // NUM_WORKERS


@functools.partial(
    pl.kernel,
    out_type=jax.ShapeDtypeStruct((B, D), jnp.int32),
    mesh=plsc.VectorSubcoreMesh(core_axis_name="c", subcore_axis_name="s"),
    compiler_params=pltpu.CompilerParams(
        use_tc_tiling_on_sc=False, needs_layout_passes=False
    ),
    scratch_types=[
        pltpu.VMEM((B_PER_W,), jnp.int32),
        pltpu.VMEM((B_PER_W, D), jnp.int32),
        pltpu.SemaphoreType.DMA,
    ],
)
def _sc_gather(idx_hbm, sum_hbm, out_hbm, idx_v, rows_v, sem):
    c = lax.axis_index("c")
    s = lax.axis_index("s")
    wid = s * 2 + c
    base = pl.multiple_of(wid * B_PER_W, B_PER_W)
    pltpu.sync_copy(idx_hbm.at[pl.ds(base, B_PER_W)], idx_v)
    pltpu.async_copy(sum_hbm.at[idx_v], rows_v, sem).wait()
    pltpu.sync_copy(rows_v, out_hbm.at[pl.ds(base, B_PER_W)])


def kernel(voc_idxs, pai_concept, pai_character):
    idx = voc_idxs.astype(jnp.int32)
    ct = jnp.transpose(pai_concept, (1, 2, 0))
    cht = jnp.transpose(pai_character, (1, 2, 0))
    sum_dv = _tc_scan(ct, cht)
    return _sc_gather(idx, sum_dv.T)


# speedup vs baseline: 5.7817x; 1.0326x over previous
import functools

import jax
import jax.numpy as jnp
from jax import lax
from jax.experimental import pallas as pl
from jax.experimental.pallas import tpu as pltpu
from jax.experimental.pallas import tpu_sc as plsc

VOC = 100000
D = 16
K = 32
B = 4096


VB = 4096
GRID = (VOC + VB - 1) // VB


def _tc_body(cref, chref, oref):
    def table_argmax(ref):
        m = ref[:, 0, :]
        am = jnp.zeros(m.shape, jnp.int32)
        for k in range(1, K):
            v = ref[:, k, :]
            gt = v > m
            am = jnp.where(gt, k, am)
            m = jnp.where(gt, v, m)
        return am

    oref[...] = table_argmax(cref) + table_argmax(chref)


def _tc_scan(ct, cht):
    return pl.pallas_call(
        _tc_body,
        grid=(GRID,),
        in_specs=[
            pl.BlockSpec((D, K, VB), lambda i: (0, 0, i)),
            pl.BlockSpec((D, K, VB), lambda i: (0, 0, i)),
        ],
        out_specs=pl.BlockSpec((D, VB), lambda i: (0, i)),
        out_shape=jax.ShapeDtypeStruct((D, VOC), jnp.int32),
    )(ct, cht)


NUM_WORKERS = 32
B_PER_W = B // NUM_WORKERS


@functools.partial(
    pl.kernel,
    out_type=jax.ShapeDtypeStruct((B, D), jnp.int32),
    mesh=plsc.VectorSubcoreMesh(core_axis_name="c", subcore_axis_name="s"),
    compiler_params=pltpu.CompilerParams(
        use_tc_tiling_on_sc=False, needs_layout_passes=False
    ),
    scratch_types=[
        pltpu.VMEM((B_PER_W,), jnp.int32),
        pltpu.VMEM((B_PER_W, D), jnp.int32),
        pltpu.SemaphoreType.DMA,
    ],
)
def _sc_gather(idx_hbm, sum_hbm, out_hbm, idx_v, rows_v, sem):
    c = lax.axis_index("c")
    s = lax.axis_index("s")
    wid = s * 2 + c
    base = pl.multiple_of(wid * B_PER_W, B_PER_W)
    pltpu.sync_copy(idx_hbm.at[pl.ds(base, B_PER_W)], idx_v)
    pltpu.async_copy(sum_hbm.at[idx_v], rows_v, sem).wait()
    pltpu.sync_copy(rows_v, out_hbm.at[pl.ds(base, B_PER_W)])


def kernel(voc_idxs, pai_concept, pai_character):
    idx = voc_idxs.astype(jnp.int32)
    ct = jnp.transpose(pai_concept, (1, 2, 0))
    cht = jnp.transpose(pai_character, (1, 2, 0))
    sum_dv = _tc_scan(ct, cht)
    return _sc_gather(idx, sum_dv.T)
